# baseline (speedup 1.0000x reference)
import functools

import jax
import jax.numpy as jnp
from jax import lax
from jax.experimental import pallas as pl
from jax.experimental.pallas import tpu as pltpu
from jax.experimental.pallas import tpu_sc as plsc

N = 10000
E = 320000
EMB = 128
NH = 4
REPR = 16
HR = NH * REPR
NW = 32
CH = 128
G = CH // 16
CHUNKS = 78
EPW = CHUNKS * CH
XBASE = NW * EPW
IBUF = EPW + CH
N_PAD = 10240
RPT = N_PAD // 16


def _prep_body(x_ref, w_ref, a_ref, h_ref, asat_ref):
    h = jnp.dot(x_ref[...], w_ref[...], preferred_element_type=jnp.float32)
    h_ref[...] = h
    asat_ref[...] = jnp.dot(h, a_ref[...], preferred_element_type=jnp.float32)


_prep = pl.pallas_call(
    _prep_body,
    out_shape=(
        jax.ShapeDtypeStruct((N, HR), jnp.float32),
        jax.ShapeDtypeStruct((N, 8), jnp.float32),
    ),
)


def _combine_body(o_ref, d_ref, b_ref, out_ref):
    o = o_ref[0] + o_ref[1]
    dsum = d_ref[0] + d_ref[1]
    parts = [jnp.broadcast_to(dsum[:, hh:hh + 1], (N_PAD, REPR)) for hh in range(NH)]
    d64 = jnp.concatenate(parts, axis=1)
    out_ref[...] = o / (d64 + 1e-16) + b_ref[...]


_combine = pl.pallas_call(
    _combine_body,
    out_shape=jax.ShapeDtypeStruct((N_PAD, HR), jnp.float32),
)

_mesh = plsc.VectorSubcoreMesh(core_axis_name="c", subcore_axis_name="s")


@functools.partial(
    pl.kernel,
    out_type=(
        jax.ShapeDtypeStruct((2, N_PAD, HR), jnp.float32),
        jax.ShapeDtypeStruct((2, N_PAD, 16), jnp.float32),
    ),
    mesh=_mesh,
    compiler_params=pltpu.CompilerParams(
        needs_layout_passes=False, use_tc_tiling_on_sc=False),
    scratch_types=[
        pltpu.VMEM((IBUF,), jnp.int32),
        pltpu.VMEM((IBUF,), jnp.int32),
        pltpu.VMEM((CH, HR), jnp.float32),
        pltpu.VMEM((CH, 16), jnp.float32),
        pltpu.VMEM((CH, 8), jnp.float32),
        pltpu.VMEM((CH, 8), jnp.float32),
        pltpu.VMEM((CH, HR), jnp.float32),
        pltpu.VMEM((CH, 16), jnp.float32),
        pltpu.VMEM((CH, 8), jnp.float32),
        pltpu.VMEM((CH, 8), jnp.float32),
        pltpu.VMEM((CH, HR), jnp.float32),
        pltpu.VMEM((CH, 16), jnp.float32),
        pltpu.VMEM((CH, 8), jnp.float32),
        pltpu.VMEM((CH, 8), jnp.float32),
        pltpu.VMEM_SHARED((N_PAD, HR), jnp.float32),
        pltpu.VMEM_SHARED((N_PAD, 16), jnp.float32),
    ] + [pltpu.SemaphoreType.DMA] * 15,
)
def _edge_kernel(asat_hbm, src_hbm, dst_hbm, h_hbm, out_raw, den_raw,
                 src_all, dst_all, rows_a, exr_a, as_a, at_a,
                 rows_b, exr_b, as_b, at_b, rows_c, exr_c, as_c, at_c,
                 out_acc, den_acc,
                 gh_a, gs_a, gt_a, so_a, sd_a, gh_b, gs_b, gt_b, so_b, sd_b,
                 gh_c, gs_c, gt_c, so_c, sd_c):
    c = lax.axis_index("c")
    s = lax.axis_index("s")
    wid = c * 16 + s
    r0 = s * RPT
    iot = lax.iota(jnp.int32, 16)
    zero16 = jnp.zeros((16,), jnp.float32)

    bufs = ((rows_a, exr_a, as_a, at_a, gh_a, gs_a, gt_a, so_a, sd_a),
            (rows_b, exr_b, as_b, at_b, gh_b, gs_b, gt_b, so_b, sd_b),
            (rows_c, exr_c, as_c, at_c, gh_c, gs_c, gt_c, so_c, sd_c))

    def _zero_bufs(i, carry):
        exr_a[i, :] = zero16
        exr_b[i, :] = zero16
        exr_c[i, :] = zero16
        for k in range(NH):
            rows_a[i, pl.ds(k * 16, 16)] = zero16
        return carry

    lax.fori_loop(0, CH, _zero_bufs, 0)

    for k in range(RPT // CH):
        pltpu.sync_copy(rows_a, out_acc.at[pl.ds(r0 + k * CH, CH)])
        pltpu.sync_copy(exr_a, den_acc.at[pl.ds(r0 + k * CH, CH)])

    pltpu.sync_copy(src_hbm.at[pl.ds(wid * EPW, EPW)],
                    src_all.at[pl.ds(0, EPW)])
    pltpu.sync_copy(dst_hbm.at[pl.ds(wid * EPW, EPW)],
                    dst_all.at[pl.ds(0, EPW)])

    @pl.when(wid < 4)
    def _stage_extra():
        pltpu.sync_copy(src_hbm.at[pl.ds(XBASE + wid * CH, CH)],
                        src_all.at[pl.ds(EPW, CH)])
        pltpu.sync_copy(dst_hbm.at[pl.ds(XBASE + wid * CH, CH)],
                        dst_all.at[pl.ds(EPW, CH)])

    plsc.subcore_barrier()

    def _issue_gathers(ci, which):
        rows_v, _, as_v, at_v, gh, gs, gt, _, _ = bufs[which]
        sg = src_all.at[pl.ds(ci * CH, CH)]
        dg = dst_all.at[pl.ds(ci * CH, CH)]
        cps = (pltpu.async_copy(h_hbm.at[sg], rows_v, gh),
               pltpu.async_copy(asat_hbm.at[sg], as_v, gs),
               pltpu.async_copy(asat_hbm.at[dg], at_v, gt))
        return cps

    def _process(ci, which):
        rows_v, exr_v, as_v, at_v, gh, gs, gt, so, sd = bufs[which]
        dg = dst_all.at[pl.ds(ci * CH, CH)]
        pltpu.make_async_copy(asat_hbm.at[dg], as_v, gs).wait()
        pltpu.make_async_copy(asat_hbm.at[dg], at_v, gt).wait()

        @plsc.parallel_loop(0, G, step=1, unroll=4)
        def _logits(g):
            rowi = iot + g * 16
            for hh in range(NH):
                a_s = plsc.load_gather(
                    as_v, [rowi, jnp.full((16,), hh, jnp.int32)])
                a_t = plsc.load_gather(
                    at_v, [rowi, jnp.full((16,), 4 + hh, jnp.int32)])
                al = a_s + a_t
                al = jnp.maximum(al, al * 0.2)
                exh = jnp.exp(al)
                plsc.store_scatter(
                    exr_v, [rowi, jnp.full((16,), hh, jnp.int32)], exh)
        cp_d = pltpu.async_copy(exr_v, den_acc.at[dg], sd, add=True)
        pltpu.make_async_copy(h_hbm.at[dg], rows_v, gh).wait()

        @plsc.parallel_loop(0, CH, step=1, unroll=16)
        def _scale(e):
            exv = exr_v[e, :]
            for hh in range(NH):
                sc = exv[hh]
                rows_v[e, pl.ds(hh * 16, 16)] = rows_v[e, pl.ds(hh * 16, 16)] * sc

        cp_o = pltpu.async_copy(rows_v, out_acc.at[dg], so, add=True)
        return cp_d, cp_o

    def _wait_scatters(which):
        rows_v, exr_v, _, _, _, _, _, so, sd = bufs[which]
        dg = dst_all.at[pl.ds(0, CH)]
        pltpu.make_async_copy(exr_v, den_acc.at[dg], sd).wait()
        pltpu.make_async_copy(rows_v, out_acc.at[dg], so).wait()

    _issue_gathers(0, 0)
    _issue_gathers(1, 1)

    def _round(c0, first, last):
        cpd0, cpo0 = _process(c0, 0)
        if not first:
            _wait_scatters(2)
        _issue_gathers(c0 + 2, 2)
        cpd1, cpo1 = _process(c0 + 1, 1)
        cpd0.wait()
        cpo0.wait()
        if not last:
            _issue_gathers(c0 + 3, 0)
        cpd2, cpo2 = _process(c0 + 2, 2)
        cpd1.wait()
        cpo1.wait()
        if not last:
            _issue_gathers(c0 + 4, 1)
        if last:
            cpd2.wait()
            cpo2.wait()

    _round(0, True, False)

    def _body(t, carry):
        _round(3 * t, False, False)
        return carry

    lax.fori_loop(1, CHUNKS // 3 - 1, _body, 0)
    _round(CHUNKS - 3, False, True)

    @pl.when(wid < 4)
    def _extra():
        _issue_gathers(CHUNKS, 0)
        cpd_x, cpo_x = _process(CHUNKS, 0)
        cpd_x.wait()
        cpo_x.wait()

    plsc.subcore_barrier()
    pltpu.sync_copy(out_acc.at[pl.ds(r0, RPT)], out_raw.at[c, pl.ds(r0, RPT)])
    pltpu.sync_copy(den_acc.at[pl.ds(r0, RPT)], den_raw.at[c, pl.ds(r0, RPT)])


def kernel(x, path, W, att, bias):
    att_r = att.reshape(NH, 2 * REPR)
    eye = jnp.eye(NH, dtype=jnp.float32)
    a_src = (att_r[:, :REPR, None] * eye[:, None, :]).reshape(HR, NH)
    a_dst = (att_r[:, REPR:, None] * eye[:, None, :]).reshape(HR, NH)
    A = jnp.concatenate([a_src, a_dst], axis=1)

    h, asat = _prep(x, W, A)
    out_raw, den_raw = _edge_kernel(asat, path[0], path[1], h)
    return _combine(out_raw, den_raw, bias.reshape(1, HR))[:N]


# speedup vs baseline: 1.0766x; 1.0766x over previous
import functools

import jax
import jax.numpy as jnp
from jax import lax
from jax.experimental import pallas as pl
from jax.experimental.pallas import tpu as pltpu
from jax.experimental.pallas import tpu_sc as plsc

N = 10000
E = 320000
EMB = 128
NH = 4
REPR = 16
HR = NH * REPR
NW = 32
CH = 128
G = CH // 16
CHUNKS = 78
EPW = CHUNKS * CH
XBASE = NW * EPW
IBUF = EPW + CH
N_PAD = 10240
RPT = N_PAD // 16


def _prep_body(x_ref, w_ref, a_ref, h_ref, asat_ref):
    h = jnp.dot(x_ref[...], w_ref[...], preferred_element_type=jnp.float32)
    h_ref[...] = h
    asat_ref[...] = jnp.dot(h, a_ref[...], preferred_element_type=jnp.float32)


_prep = pl.pallas_call(
    _prep_body,
    out_shape=(
        jax.ShapeDtypeStruct((N, HR), jnp.float32),
        jax.ShapeDtypeStruct((N, 8), jnp.float32),
    ),
)


def _combine_body(o_ref, d_ref, b_ref, out_ref):
    o = o_ref[0, :N] + o_ref[1, :N]
    dsum = d_ref[0, :N] + d_ref[1, :N]
    parts = [jnp.broadcast_to(dsum[:, hh:hh + 1], (N, REPR)) for hh in range(NH)]
    d64 = jnp.concatenate(parts, axis=1)
    out_ref[...] = o / (d64 + 1e-16) + b_ref[...]


_combine = pl.pallas_call(
    _combine_body,
    out_shape=jax.ShapeDtypeStruct((N, HR), jnp.float32),
)

_mesh = plsc.VectorSubcoreMesh(core_axis_name="c", subcore_axis_name="s")


@functools.partial(
    pl.kernel,
    out_type=(
        jax.ShapeDtypeStruct((2, N_PAD, HR), jnp.float32),
        jax.ShapeDtypeStruct((2, N_PAD, 16), jnp.float32),
    ),
    mesh=_mesh,
    compiler_params=pltpu.CompilerParams(
        needs_layout_passes=False, use_tc_tiling_on_sc=False),
    scratch_types=[
        pltpu.VMEM((IBUF,), jnp.int32),
        pltpu.VMEM((IBUF,), jnp.int32),
        pltpu.VMEM((CH, HR), jnp.float32),
        pltpu.VMEM((CH, 16), jnp.float32),
        pltpu.VMEM((CH, 8), jnp.float32),
        pltpu.VMEM((CH, 8), jnp.float32),
        pltpu.VMEM((CH, HR), jnp.float32),
        pltpu.VMEM((CH, 16), jnp.float32),
        pltpu.VMEM((CH, 8), jnp.float32),
        pltpu.VMEM((CH, 8), jnp.float32),
        pltpu.VMEM((CH, HR), jnp.float32),
        pltpu.VMEM((CH, 16), jnp.float32),
        pltpu.VMEM((CH, 8), jnp.float32),
        pltpu.VMEM((CH, 8), jnp.float32),
        pltpu.VMEM_SHARED((N_PAD, HR), jnp.float32),
        pltpu.VMEM_SHARED((N_PAD, 16), jnp.float32),
    ] + [pltpu.SemaphoreType.DMA] * 15,
)
def _edge_kernel(asat_hbm, src_hbm, dst_hbm, h_hbm, out_raw, den_raw,
                 src_all, dst_all, rows_a, exr_a, as_a, at_a,
                 rows_b, exr_b, as_b, at_b, rows_c, exr_c, as_c, at_c,
                 out_acc, den_acc,
                 gh_a, gs_a, gt_a, so_a, sd_a, gh_b, gs_b, gt_b, so_b, sd_b,
                 gh_c, gs_c, gt_c, so_c, sd_c):
    c = lax.axis_index("c")
    s = lax.axis_index("s")
    wid = c * 16 + s
    r0 = s * RPT
    iot = lax.iota(jnp.int32, 16)
    zero16 = jnp.zeros((16,), jnp.float32)

    bufs = ((rows_a, exr_a, as_a, at_a, gh_a, gs_a, gt_a, so_a, sd_a),
            (rows_b, exr_b, as_b, at_b, gh_b, gs_b, gt_b, so_b, sd_b),
            (rows_c, exr_c, as_c, at_c, gh_c, gs_c, gt_c, so_c, sd_c))

    def _zero_bufs(i, carry):
        exr_a[i, :] = zero16
        exr_b[i, :] = zero16
        exr_c[i, :] = zero16
        for k in range(NH):
            rows_a[i, pl.ds(k * 16, 16)] = zero16
        return carry

    lax.fori_loop(0, CH, _zero_bufs, 0)

    for k in range(RPT // CH):
        pltpu.sync_copy(rows_a, out_acc.at[pl.ds(r0 + k * CH, CH)])
        pltpu.sync_copy(exr_a, den_acc.at[pl.ds(r0 + k * CH, CH)])

    pltpu.sync_copy(src_hbm.at[pl.ds(wid * EPW, EPW)],
                    src_all.at[pl.ds(0, EPW)])
    pltpu.sync_copy(dst_hbm.at[pl.ds(wid * EPW, EPW)],
                    dst_all.at[pl.ds(0, EPW)])

    @pl.when(wid < 4)
    def _stage_extra():
        pltpu.sync_copy(src_hbm.at[pl.ds(XBASE + wid * CH, CH)],
                        src_all.at[pl.ds(EPW, CH)])
        pltpu.sync_copy(dst_hbm.at[pl.ds(XBASE + wid * CH, CH)],
                        dst_all.at[pl.ds(EPW, CH)])

    plsc.subcore_barrier()

    def _issue_gathers(ci, which):
        rows_v, _, as_v, at_v, gh, gs, gt, _, _ = bufs[which]
        sg = src_all.at[pl.ds(ci * CH, CH)]
        dg = dst_all.at[pl.ds(ci * CH, CH)]
        cps = (pltpu.async_copy(h_hbm.at[sg], rows_v, gh),
               pltpu.async_copy(asat_hbm.at[sg], as_v, gs),
               pltpu.async_copy(asat_hbm.at[dg], at_v, gt))
        return cps

    def _process(ci, which):
        rows_v, exr_v, as_v, at_v, gh, gs, gt, so, sd = bufs[which]
        dg = dst_all.at[pl.ds(ci * CH, CH)]
        pltpu.make_async_copy(asat_hbm.at[dg], as_v, gs).wait()
        pltpu.make_async_copy(asat_hbm.at[dg], at_v, gt).wait()

        @plsc.parallel_loop(0, G, step=1, unroll=2)
        def _logits(g):
            rowi = iot + g * 16
            for hh in range(NH):
                a_s = plsc.load_gather(
                    as_v, [rowi, jnp.full((16,), hh, jnp.int32)])
                a_t = plsc.load_gather(
                    at_v, [rowi, jnp.full((16,), 4 + hh, jnp.int32)])
                al = a_s + a_t
                al = jnp.maximum(al, al * 0.2)
                exh = jnp.exp(al)
                plsc.store_scatter(
                    exr_v, [rowi, jnp.full((16,), hh, jnp.int32)], exh)
        cp_d = pltpu.async_copy(exr_v, den_acc.at[dg], sd, add=True)
        pltpu.make_async_copy(h_hbm.at[dg], rows_v, gh).wait()

        @plsc.parallel_loop(0, CH, step=1, unroll=8)
        def _scale(e):
            exv = exr_v[e, :]
            for hh in range(NH):
                sc = exv[hh]
                rows_v[e, pl.ds(hh * 16, 16)] = rows_v[e, pl.ds(hh * 16, 16)] * sc

        cp_o = pltpu.async_copy(rows_v, out_acc.at[dg], so, add=True)
        return cp_d, cp_o

    def _wait_scatters(which):
        rows_v, exr_v, _, _, _, _, _, so, sd = bufs[which]
        dg = dst_all.at[pl.ds(0, CH)]
        pltpu.make_async_copy(exr_v, den_acc.at[dg], sd).wait()
        pltpu.make_async_copy(rows_v, out_acc.at[dg], so).wait()

    _issue_gathers(0, 0)
    _issue_gathers(1, 1)

    def _round(c0, first, last):
        cpd0, cpo0 = _process(c0, 0)
        if not first:
            _wait_scatters(2)
        _issue_gathers(c0 + 2, 2)
        cpd1, cpo1 = _process(c0 + 1, 1)
        cpd0.wait()
        cpo0.wait()
        if not last:
            _issue_gathers(c0 + 3, 0)
        cpd2, cpo2 = _process(c0 + 2, 2)
        cpd1.wait()
        cpo1.wait()
        if not last:
            _issue_gathers(c0 + 4, 1)
        if last:
            cpd2.wait()
            cpo2.wait()

    _round(0, True, False)

    def _body(t, carry):
        _round(3 * t, False, False)
        return carry

    lax.fori_loop(1, CHUNKS // 3 - 1, _body, 0)
    _round(CHUNKS - 3, False, True)

    @pl.when(wid < 4)
    def _extra():
        _issue_gathers(CHUNKS, 0)
        cpd_x, cpo_x = _process(CHUNKS, 0)
        cpd_x.wait()
        cpo_x.wait()

    plsc.subcore_barrier()
    pltpu.sync_copy(out_acc.at[pl.ds(r0, RPT)], out_raw.at[c, pl.ds(r0, RPT)])
    pltpu.sync_copy(den_acc.at[pl.ds(r0, RPT)], den_raw.at[c, pl.ds(r0, RPT)])


def kernel(x, path, W, att, bias):
    att_r = att.reshape(NH, 2 * REPR)
    eye = jnp.eye(NH, dtype=jnp.float32)
    a_src = (att_r[:, :REPR, None] * eye[:, None, :]).reshape(HR, NH)
    a_dst = (att_r[:, REPR:, None] * eye[:, None, :]).reshape(HR, NH)
    A = jnp.concatenate([a_src, a_dst], axis=1)

    h, asat = _prep(x, W, A)
    out_raw, den_raw = _edge_kernel(asat, path[0], path[1], h)
    return _combine(out_raw, den_raw, bias.reshape(1, HR))
